# baseline (speedup 1.0000x reference)
import jax
import jax.numpy as jnp
from jax.experimental import pallas as pl
from jax.experimental.pallas import tpu as pltpu

NB = 11
P = 4
H = 512
W = 512
HO = 127
WO = 127
HC = HO * P
CH = 32
NCHUNK = H // CH

_ATAN_COEFFS = tuple(
    c * NB / float(jnp.pi) for c in (
        1.0, -0.33333278, 0.19998075, -0.14260016, 0.10932341,
        -0.08349725, 0.057089556, -0.030351864, 0.01048765, -0.0017011701,
    )
)


def _hog_kernel(x_ref, o_ref, m_ref, gx_ref, gy_ref):
    xb = x_ref[0, 0]

    a_ = jax.lax.broadcasted_iota(jnp.int32, (W, W), 0)
    j_ = jax.lax.broadcasted_iota(jnp.int32, (W, W), 1)
    wlo = jnp.where(a_ == j_ - 1, jnp.float32(1.0), jnp.float32(0.0))
    whi = jnp.where(a_ == j_ + 1, jnp.float32(1.0), jnp.float32(0.0))
    dmat = wlo - whi
    emat = whi - wlo
    gx_ref[...] = jax.lax.dot(xb, dmat)
    gy_ref[...] = jax.lax.dot(emat, xb)

    def chunk(i, carry):
        gxc = gx_ref[pl.ds(i * CH, CH), :]
        gyc = gy_ref[pl.ds(i * CH, CH), :]

        ay = jnp.abs(gxc)
        ax = jnp.abs(gyc)
        mn = jnp.minimum(ay, ax)
        mx = jnp.maximum(ay, ax)
        t = mn / jnp.where(mx > 0, mx, jnp.float32(1.0))
        u = t * t
        acc = jnp.full_like(u, _ATAN_COEFFS[-1])
        for cf in _ATAN_COEFFS[-2::-1]:
            acc = acc * u + jnp.float32(cf)
        q = acc * t
        base = jnp.where(ay > ax, jnp.float32(0.5 * NB) - q, q)
        base = jnp.where(gyc < 0, jnp.float32(NB) - base, base)
        tt = jnp.where(gxc < 0, -base, base)

        f = jnp.floor(tt)
        c = jnp.ceil(tt)
        f = jnp.where(f < 0, f + NB, f)
        f = jnp.where(f >= NB, f - NB, f)
        c = jnp.where(c < 0, c + NB, c)
        c = jnp.where(c >= NB, c - NB, c)

        s = ay * ay + ax * ax
        mag = jnp.where(s > 0, s * jax.lax.rsqrt(s), 0.0)

        fb = f.astype(jnp.bfloat16)
        cb = c.astype(jnp.bfloat16)
        magb = mag.astype(jnp.bfloat16)
        omb = (1.0 - mag).astype(jnp.bfloat16)

        zb = jnp.bfloat16(0.0)
        for k in range(NB):
            fk = jnp.bfloat16(k)
            mk = (jnp.where(fb == fk, magb, zb)
                  + jnp.where(cb == fk, omb, zb))
            m_ref[pl.ds(k * H + i * CH, CH), :] = mk
        return carry

    jax.lax.fori_loop(0, NCHUNK, chunk, 0, unroll=2)

    row = jax.lax.broadcasted_iota(jnp.int32, (W, WO), 0)
    col = jax.lax.broadcasted_iota(jnp.int32, (W, WO), 1)
    pw = jnp.where((row - 1) // P == col, jnp.float32(0.25), jnp.float32(0.0))
    pw = jnp.where((row >= 1) & (row <= HC), pw, jnp.float32(0.0))
    pwb = pw.astype(jnp.bfloat16)
    rowt = jax.lax.broadcasted_iota(jnp.int32, (HO, H), 0)
    colt = jax.lax.broadcasted_iota(jnp.int32, (HO, H), 1)
    pht = jnp.where((colt - 1) // P == rowt, jnp.float32(0.25), jnp.float32(0.0))
    pht = jnp.where((colt >= 1) & (colt <= HC), pht, jnp.float32(0.0))

    a_all = jax.lax.dot(m_ref[...], pwb,
                        preferred_element_type=jnp.float32)
    for k in range(NB):
        o_ref[0, k] = jax.lax.dot(pht, a_all[k * H:(k + 1) * H, :])


def _hog_batch(x):
    n = x.shape[0]
    return pl.pallas_call(
        _hog_kernel,
        grid=(n,),
        in_specs=[pl.BlockSpec((1, 1, H, W), lambda b: (b, 0, 0, 0))],
        out_specs=pl.BlockSpec((1, NB, HO, WO), lambda b: (b, 0, 0, 0)),
        out_shape=jax.ShapeDtypeStruct((n, NB, HO, WO), jnp.float32),
        scratch_shapes=[pltpu.VMEM((NB * H, W), jnp.bfloat16),
                        pltpu.VMEM((H, W), jnp.float32),
                        pltpu.VMEM((H, W), jnp.float32)],
    )(x)


def kernel(x):
    return _hog_batch(x)


# speedup vs baseline: 1.0405x; 1.0405x over previous
import jax
import jax.numpy as jnp
from jax.experimental import pallas as pl
from jax.experimental.pallas import tpu as pltpu

NB = 11
P = 4
H = 512
W = 512
HO = 127
WO = 127
HC = HO * P
CH = 16
NCHUNK = H // CH

_ATAN_COEFFS = tuple(
    c * NB / float(jnp.pi) for c in (
        0.999999226, -0.33325678, 0.198720403, -0.134478641,
        0.083126453, -0.0363604309, 0.00764835393,
    )
)


def _hog_kernel(x_ref, o_ref, m_ref, gx_ref, gy_ref):
    xb = x_ref[0, 0]

    a_ = jax.lax.broadcasted_iota(jnp.int32, (W, W), 0)
    j_ = jax.lax.broadcasted_iota(jnp.int32, (W, W), 1)
    wlo = jnp.where(a_ == j_ - 1, jnp.float32(1.0), jnp.float32(0.0))
    whi = jnp.where(a_ == j_ + 1, jnp.float32(1.0), jnp.float32(0.0))
    dmat = wlo - whi
    emat = whi - wlo
    gx_ref[...] = jax.lax.dot(xb, dmat)
    gy_ref[...] = jax.lax.dot(emat, xb)

    def chunk(i, carry):
        gxc = gx_ref[pl.ds(i * CH, CH), :]
        gyc = gy_ref[pl.ds(i * CH, CH), :]

        ay = jnp.abs(gxc)
        ax = jnp.abs(gyc)
        mn = jnp.minimum(ay, ax)
        mx = jnp.maximum(ay, ax)
        t = mn / jnp.where(mx > 0, mx, jnp.float32(1.0))
        u = t * t
        acc = jnp.full_like(u, _ATAN_COEFFS[-1])
        for cf in _ATAN_COEFFS[-2::-1]:
            acc = acc * u + jnp.float32(cf)
        q = acc * t
        base = jnp.where(ay > ax, jnp.float32(0.5 * NB) - q, q)
        base = jnp.where(gyc < 0, jnp.float32(NB) - base, base)
        tt = jnp.where(gxc < 0, -base, base)

        f = jnp.floor(tt)
        c = jnp.ceil(tt)
        f = jnp.where(f < 0, f + NB, f)
        f = jnp.where(f >= NB, f - NB, f)
        c = jnp.where(c < 0, c + NB, c)
        c = jnp.where(c >= NB, c - NB, c)

        s = ay * ay + ax * ax
        mag = jnp.where(s > 0, s * jax.lax.rsqrt(s), 0.0)

        fb = f.astype(jnp.bfloat16)
        cb = c.astype(jnp.bfloat16)
        magb = mag.astype(jnp.bfloat16)
        omb = (1.0 - mag).astype(jnp.bfloat16)

        zb = jnp.bfloat16(0.0)
        for k in range(NB):
            fk = jnp.bfloat16(k)
            mk = (jnp.where(fb == fk, magb, zb)
                  + jnp.where(cb == fk, omb, zb))
            m_ref[pl.ds(k * H + i * CH, CH), :] = mk
        return carry

    jax.lax.fori_loop(0, NCHUNK, chunk, 0, unroll=2)

    row = jax.lax.broadcasted_iota(jnp.int32, (W, WO), 0)
    col = jax.lax.broadcasted_iota(jnp.int32, (W, WO), 1)
    pw = jnp.where((row - 1) // P == col, jnp.float32(0.25), jnp.float32(0.0))
    pw = jnp.where((row >= 1) & (row <= HC), pw, jnp.float32(0.0))
    pwb = pw.astype(jnp.bfloat16)
    rowt = jax.lax.broadcasted_iota(jnp.int32, (HO, H), 0)
    colt = jax.lax.broadcasted_iota(jnp.int32, (HO, H), 1)
    pht = jnp.where((colt - 1) // P == rowt, jnp.float32(0.25), jnp.float32(0.0))
    pht = jnp.where((colt >= 1) & (colt <= HC), pht, jnp.float32(0.0))

    a_all = jax.lax.dot(m_ref[...], pwb,
                        preferred_element_type=jnp.float32)
    for k in range(NB):
        o_ref[0, k] = jax.lax.dot(pht, a_all[k * H:(k + 1) * H, :])


def _hog_batch(x):
    n = x.shape[0]
    return pl.pallas_call(
        _hog_kernel,
        grid=(n,),
        in_specs=[pl.BlockSpec((1, 1, H, W), lambda b: (b, 0, 0, 0))],
        out_specs=pl.BlockSpec((1, NB, HO, WO), lambda b: (b, 0, 0, 0)),
        out_shape=jax.ShapeDtypeStruct((n, NB, HO, WO), jnp.float32),
        scratch_shapes=[pltpu.VMEM((NB * H, W), jnp.bfloat16),
                        pltpu.VMEM((H, W), jnp.float32),
                        pltpu.VMEM((H, W), jnp.float32)],
    )(x)


def kernel(x):
    return _hog_batch(x)
